# baseline (speedup 1.0000x reference)
import functools

import jax
import jax.numpy as jnp
from jax import lax
from jax.experimental import pallas as pl
from jax.experimental.pallas import tpu as pltpu
from jax.experimental.pallas import tpu_sc as plsc

B = 16384
L = 20
EMB = 32
VOCAB = 1000000
B_TOT = B * L
TAB_R = VOCAB // 4
OUT_R = B_TOT // 4

_info = plsc.get_sparse_core_info()
_NC = _info.num_cores
_NS = _info.num_subcores
NW = _NC * _NS
B_PER_W = B_TOT // NW
CHUNK = 320
N_CHUNKS = B_PER_W // CHUNK
G16 = CHUNK // 16
C128 = CHUNK // 4
N_BUF = 2

_mesh = plsc.VectorSubcoreMesh(core_axis_name="c", subcore_axis_name="s")


@functools.partial(
    pl.kernel,
    mesh=_mesh,
    out_type=jax.ShapeDtypeStruct((OUT_R, 128), jnp.float32),
    scratch_types=[
        pltpu.VMEM((B_PER_W,), jnp.int32),
        pltpu.VMEM((N_BUF, CHUNK), jnp.int32),
        pltpu.VMEM((N_BUF, CHUNK, 128), jnp.float32),
        pltpu.VMEM((N_BUF, C128, 128), jnp.float32),
        pltpu.SemaphoreType.DMA((N_BUF,)),
        pltpu.SemaphoreType.DMA((N_BUF,)),
    ],
    compiler_params=pltpu.CompilerParams(
        use_tc_tiling_on_sc=False, needs_layout_passes=False),
)
def _gather_kernel(idx_hbm, table_hbm, out_hbm, idx_v, sup_v, super_v,
                   outr_v, gsem, osem):
    wid = lax.axis_index("s") * _NC + lax.axis_index("c")
    base128 = pl.multiple_of(wid * (B_PER_W // 4), C128)

    pltpu.sync_copy(idx_hbm.at[wid], idx_v)

    def compute_sup(g, b):
        def body(i, carry):
            v = idx_v[pl.ds(g * CHUNK + i * 16, 16)]
            sup_v.at[b][pl.ds(i * 16, 16)] = lax.shift_right_logical(v, 2)
            return carry
        lax.fori_loop(0, G16, body, 0)

    def fire_gather(b):
        return pltpu.async_copy(
            table_hbm.at[sup_v.at[b]], super_v.at[b], gsem.at[b])

    def selection(g, b):
        sup_b = super_v.at[b]
        out_b = outr_v.at[b]

        def group(i, carry):
            idxv = idx_v[pl.ds(g * CHUNK + i * 16, 16)]
            w32 = lax.shift_left(jnp.bitwise_and(idxv, 3), 5)
            rvec = lax.iota(jnp.int32, 16) + i * 16
            rv32 = lax.shift_left(rvec, 5)
            for c in range(EMB):
                src_c = w32 + c
                vals = plsc.load_gather(sup_b, [rvec, src_c])
                flat = rv32 + c
                qv = lax.shift_right_logical(flat, 7)
                cv = jnp.bitwise_and(flat, 127)
                plsc.store_scatter(out_b, [qv, cv], vals)
            return carry
        lax.fori_loop(0, G16, group, 0)

    def out_slice(g):
        return out_hbm.at[pl.ds(base128 + g * C128, C128)]

    for b in range(N_BUF):
        compute_sup(b, b)
        fire_gather(b)

    def chunk_step(g0, carry):
        for b in range(N_BUF):
            g = g0 + b
            pltpu.make_async_copy(
                table_hbm.at[sup_v.at[b]], super_v.at[b], gsem.at[b]).wait()

            @pl.when(g >= N_BUF)
            def _wait_prev_store():
                pltpu.make_async_copy(
                    outr_v.at[b], out_slice(g - N_BUF), osem.at[b]).wait()

            selection(g, b)
            pltpu.async_copy(outr_v.at[b], out_slice(g), osem.at[b])

            @pl.when(g + N_BUF < N_CHUNKS)
            def _fire_next():
                compute_sup(g + N_BUF, b)
                fire_gather(b)
        return carry

    lax.fori_loop(0, N_CHUNKS // N_BUF, lambda i, c: chunk_step(i * N_BUF, c),
                  0)

    for b in range(N_BUF):
        pltpu.make_async_copy(
            outr_v.at[b], out_slice(N_CHUNKS - N_BUF + b), osem.at[b]).wait()


def kernel(x, table):
    idx = x.reshape(NW, B_PER_W)
    table128 = table.reshape(TAB_R, 128)
    out = _gather_kernel(idx, table128)
    return out.reshape(B, L, EMB)


# speedup vs baseline: 1.0331x; 1.0331x over previous
import functools

import jax
import jax.numpy as jnp
from jax import lax
from jax.experimental import pallas as pl
from jax.experimental.pallas import tpu as pltpu
from jax.experimental.pallas import tpu_sc as plsc

B = 16384
L = 20
EMB = 32
B_TOT = B * L

_info = plsc.get_sparse_core_info()
_NC = _info.num_cores
_NS = _info.num_subcores
NW = _NC * _NS
B_PER_W = B_TOT // NW
CHUNK = 1280
N_STEPS = B_PER_W // CHUNK
N_BUF = 2
N_SUB = 4
SUB = CHUNK // N_SUB

_mesh = plsc.VectorSubcoreMesh(core_axis_name="c", subcore_axis_name="s")


@functools.partial(
    pl.kernel,
    mesh=_mesh,
    out_type=jax.ShapeDtypeStruct((B_TOT, EMB), jnp.float32),
    scratch_types=[
        pltpu.VMEM((N_STEPS * N_SUB, SUB), jnp.int32),
        pltpu.VMEM((N_BUF, CHUNK, EMB), jnp.float32),
        pltpu.SemaphoreType.DMA((N_BUF,)),
        pltpu.SemaphoreType.DMA((N_BUF,)),
    ],
    compiler_params=pltpu.CompilerParams(use_tc_tiling_on_sc=False),
)
def _gather_kernel(idx_hbm, table_hbm, out_hbm, idx_v, rows_v, gsem, osem):
    wid = lax.axis_index("s") * _NC + lax.axis_index("c")
    base = pl.multiple_of(wid * B_PER_W, CHUNK)

    pltpu.sync_copy(idx_hbm.at[wid], idx_v)

    def fire(g, b):
        return [
            pltpu.async_copy(
                table_hbm.at[idx_v.at[g * N_SUB + s]],
                rows_v.at[b, pl.ds(s * SUB, SUB)],
                gsem.at[b])
            for s in range(N_SUB)
        ]

    gathers = [None] * N_BUF
    stores = [None] * N_BUF
    gathers[0] = fire(0, 0)
    for g in range(N_STEPS):
        b = g % N_BUF
        if g + 1 < N_STEPS:
            nb = (g + 1) % N_BUF
            if stores[nb] is not None:
                stores[nb].wait()
            gathers[nb] = fire(g + 1, nb)
        for cp in gathers[b]:
            cp.wait()
        stores[b] = pltpu.async_copy(
            rows_v.at[b],
            out_hbm.at[pl.ds(base + g * CHUNK, CHUNK)],
            osem.at[b])
    for s in stores:
        if s is not None:
            s.wait()


def kernel(x, table):
    idx = x.reshape(NW, N_STEPS * N_SUB, SUB)
    rt_zero = jnp.where(x[0, 0] >= 0, jnp.float32(0), jnp.float32(1))
    out = _gather_kernel(idx, table + rt_zero)
    return out.reshape(B, L, EMB)


# speedup vs baseline: 1.4889x; 1.4411x over previous
import functools

import jax
import jax.numpy as jnp
from jax import lax
from jax.experimental import pallas as pl
from jax.experimental.pallas import tpu as pltpu
from jax.experimental.pallas import tpu_sc as plsc

B = 16384
L = 20
EMB = 32
B_TOT = B * L

_info = plsc.get_sparse_core_info()
_NC = _info.num_cores
_NS = _info.num_subcores
NW = _NC * _NS
B_PER_W = B_TOT // NW
CHUNK = 1280
N_STEPS = B_PER_W // CHUNK
N_BUF = 2
N_SUB = 1
SUB = CHUNK // N_SUB

_mesh = plsc.VectorSubcoreMesh(core_axis_name="c", subcore_axis_name="s")


@functools.partial(
    pl.kernel,
    mesh=_mesh,
    out_type=jax.ShapeDtypeStruct((B_TOT, EMB), jnp.float32),
    scratch_types=[
        pltpu.VMEM((N_STEPS * N_SUB, SUB), jnp.int32),
        pltpu.VMEM((N_BUF, CHUNK, EMB), jnp.float32),
        pltpu.SemaphoreType.DMA((N_BUF,)),
        pltpu.SemaphoreType.DMA((N_BUF,)),
    ],
    compiler_params=pltpu.CompilerParams(use_tc_tiling_on_sc=False),
)
def _gather_kernel(idx_hbm, table_hbm, out_hbm, idx_v, rows_v, gsem, osem):
    wid = lax.axis_index("s") * _NC + lax.axis_index("c")
    base = pl.multiple_of(wid * B_PER_W, CHUNK)

    pltpu.sync_copy(idx_hbm.at[wid], idx_v)

    def fire(g, b):
        return [
            pltpu.async_copy(
                table_hbm.at[idx_v.at[g * N_SUB + s]],
                rows_v.at[b, pl.ds(s * SUB, SUB)],
                gsem.at[b])
            for s in range(N_SUB)
        ]

    gathers = [None] * N_BUF
    stores = [None] * N_BUF
    gathers[0] = fire(0, 0)
    for g in range(N_STEPS):
        b = g % N_BUF
        if g + 1 < N_STEPS:
            nb = (g + 1) % N_BUF
            if stores[nb] is not None:
                stores[nb].wait()
            gathers[nb] = fire(g + 1, nb)
        for cp in gathers[b]:
            cp.wait()
        stores[b] = pltpu.async_copy(
            rows_v.at[b],
            out_hbm.at[pl.ds(base + g * CHUNK, CHUNK)],
            osem.at[b])
    for s in stores:
        if s is not None:
            s.wait()


def kernel(x, table):
    idx = x.reshape(NW, N_STEPS * N_SUB, SUB)
    out = _gather_kernel(idx, table)
    return out.reshape(B, L, EMB)


# speedup vs baseline: 1.7811x; 1.1963x over previous
import functools

import jax
import jax.numpy as jnp
from jax import lax
from jax.experimental import pallas as pl
from jax.experimental.pallas import tpu as pltpu
from jax.experimental.pallas import tpu_sc as plsc

B = 16384
L = 20
EMB = 32
B_TOT = B * L
OUT_R = B_TOT // 4

_info = plsc.get_sparse_core_info()
_NC = _info.num_cores
_NS = _info.num_subcores
NW = _NC * _NS
B_PER_W = B_TOT // NW
XROWS_W = B_PER_W // L
CH = 320
XR = CH // L
QD = CH // 4
N_CH = B_PER_W // CH
N_BUF = 2

_mesh = plsc.VectorSubcoreMesh(core_axis_name="c", subcore_axis_name="s")


@functools.partial(
    pl.kernel,
    mesh=_mesh,
    out_type=jax.ShapeDtypeStruct((OUT_R, 128), jnp.float32),
    scratch_types=[
        pltpu.VMEM((N_BUF, XR, L), jnp.int32),
        pltpu.VMEM((N_BUF, CH, EMB), jnp.float32),
        pltpu.VMEM((N_BUF, QD, 128), jnp.float32),
        pltpu.SemaphoreType.DMA((N_BUF,)),
        pltpu.SemaphoreType.DMA((N_BUF,)),
    ],
    compiler_params=pltpu.CompilerParams(use_tc_tiling_on_sc=True),
)
def _gather_kernel(x_hbm, table_hbm, out_hbm, xv, rowbuf, outbuf, gsem, osem):
    wid = lax.axis_index("s") * _NC + lax.axis_index("c")
    xrow0 = pl.multiple_of(wid * XROWS_W, XR)
    qrow0 = pl.multiple_of(wid * (B_PER_W // 4), QD)

    def out_slice(g):
        return out_hbm.at[pl.ds(qrow0 + g * QD, QD)]

    def issue_rows(g, b):
        pltpu.sync_copy(x_hbm.at[pl.ds(xrow0 + g * XR, XR)], xv.at[b])

        def row_body(xr, carry):
            va = xv[b, xr, pl.ds(0, 16)]
            vb = xv[b, xr, pl.ds(L - 16, 16)]
            for c in range(L):
                idx_s = va[c] if c < 16 else vb[c - (L - 16)]
                pltpu.async_copy(
                    table_hbm.at[pl.ds(idx_s, 1)],
                    rowbuf.at[b, pl.ds(xr * L + c, 1)],
                    gsem.at[b])
            return carry

        lax.fori_loop(0, XR, row_body, 0)

    def drain_rows(b):
        pltpu.make_async_copy(
            table_hbm.at[pl.ds(0, CH)], rowbuf.at[b], gsem.at[b]).wait()

    def repack(b):
        def quad_body(q, carry):
            for k in range(4):
                r = q * 4 + k
                lo = rowbuf[b, r, pl.ds(0, 16)]
                hi = rowbuf[b, r, pl.ds(16, 16)]
                outbuf.at[b][q, pl.ds(k * EMB, 16)] = lo
                outbuf.at[b][q, pl.ds(k * EMB + 16, 16)] = hi
            return carry

        lax.fori_loop(0, QD, quad_body, 0)

    for b in range(N_BUF):
        issue_rows(b, b)

    def pair_step(gp, carry):
        for b in range(N_BUF):
            g = gp * N_BUF + b
            drain_rows(b)

            @pl.when(g >= N_BUF)
            def _wait_prev_store():
                pltpu.make_async_copy(
                    outbuf.at[b], out_slice(g - N_BUF), osem.at[b]).wait()

            repack(b)
            pltpu.async_copy(outbuf.at[b], out_slice(g), osem.at[b])

            @pl.when(g + N_BUF < N_CH)
            def _issue_next():
                issue_rows(g + N_BUF, b)
        return carry

    lax.fori_loop(0, N_CH // N_BUF, pair_step, 0)

    for b in range(N_BUF):
        pltpu.make_async_copy(
            outbuf.at[b], out_slice(N_CH - N_BUF + b), osem.at[b]).wait()


def kernel(x, table):
    out = _gather_kernel(x, table)
    return out.reshape(B, L, EMB)
